# baseline (speedup 1.0000x reference)
import functools

import jax
import jax.numpy as jnp
from jax.experimental import pallas as pl
from jax.experimental.pallas import tpu as pltpu

H = 128
LN2 = 0.6931471805599453


def _spk(x):
    return jnp.maximum(x, 0.0) + jnp.log1p(jnp.exp(-jnp.abs(x))) - LN2


def _cutoff_body(r_ref, out_ref):
    r = r_ref[...]
    rc = jnp.clip(r, 0.0, 1.0)
    out_ref[...] = jnp.where(r < 1.0, 0.5 * (jnp.cos(jnp.pi * rc) + 1.0), 0.0)


def _cutoff_tc(r2d, block):
    e = r2d.shape[0]
    return pl.pallas_call(
        _cutoff_body,
        grid=(e // block,),
        in_specs=[pl.BlockSpec((block, 1), lambda i: (i, 0))],
        out_specs=pl.BlockSpec((block, 1), lambda i: (i, 0)),
        out_shape=jax.ShapeDtypeStruct((e, 1), jnp.float32),
    )(r2d)


def _edge_msg_body(r_ref, cut_ref, hsrc_ref, e_ref, w1_ref, b1_ref, w2_ref,
                   b2_ref, out_ref, *, lo, hi):
    r = r_ref[...]
    j = jax.lax.broadcasted_iota(jnp.int32, (1, H), 1).astype(jnp.float32)
    mu = lo + j * ((hi - lo) / (H - 1))
    bf = jnp.exp(-10.0 * (r - mu) ** 2) * cut_ref[...]
    t = _spk(jnp.dot(bf, w1_ref[...], preferred_element_type=jnp.float32)
             + b1_ref[...])
    filt = jnp.dot(t, w2_ref[...], preferred_element_type=jnp.float32) \
        + b2_ref[...]
    out_ref[...] = (hsrc_ref[...] + e_ref[...]) * filt


def _edge_msg_tc(r2d, cut2d, hsrc, e, w1, b1, w2, b2, lo, hi, block):
    n = r2d.shape[0]
    body = functools.partial(_edge_msg_body, lo=lo, hi=hi)
    return pl.pallas_call(
        body,
        grid=(n // block,),
        in_specs=[
            pl.BlockSpec((block, 1), lambda i: (i, 0)),
            pl.BlockSpec((block, 1), lambda i: (i, 0)),
            pl.BlockSpec((block, H), lambda i: (i, 0)),
            pl.BlockSpec((block, H), lambda i: (i, 0)),
            pl.BlockSpec((H, H), lambda i: (0, 0)),
            pl.BlockSpec((1, H), lambda i: (0, 0)),
            pl.BlockSpec((H, H), lambda i: (0, 0)),
            pl.BlockSpec((1, H), lambda i: (0, 0)),
        ],
        out_specs=pl.BlockSpec((block, H), lambda i: (i, 0)),
        out_shape=jax.ShapeDtypeStruct((n, H), jnp.float32),
    )(r2d, cut2d, hsrc, e, w1, b1.reshape(1, H), w2, b2.reshape(1, H))


def _out_update_body(hn_ref, agg_ref, wo_ref, bo_ref, out_ref):
    u = jnp.dot(agg_ref[...], wo_ref[...], preferred_element_type=jnp.float32) \
        + bo_ref[...]
    out_ref[...] = hn_ref[...] + _spk(u)


def _out_update_tc(hn, agg, wo, bo, block):
    n = hn.shape[0]
    return pl.pallas_call(
        _out_update_body,
        grid=(n // block,),
        in_specs=[
            pl.BlockSpec((block, H), lambda i: (i, 0)),
            pl.BlockSpec((block, H), lambda i: (i, 0)),
            pl.BlockSpec((H, H), lambda i: (0, 0)),
            pl.BlockSpec((1, H), lambda i: (0, 0)),
        ],
        out_specs=pl.BlockSpec((block, H), lambda i: (i, 0)),
        out_shape=jax.ShapeDtypeStruct((n, H), jnp.float32),
    )(hn, agg, wo, bo.reshape(1, H))


def _readout1_body(hn_ref, w1_ref, b1_ref, w2_ref, b2_ref, out_ref):
    i = pl.program_id(0)
    t = _spk(jnp.dot(hn_ref[...], w1_ref[...],
                     preferred_element_type=jnp.float32) + b1_ref[...])
    x = jnp.dot(t, w2_ref[...], preferred_element_type=jnp.float32) \
        + b2_ref[...]
    part = jnp.sum(x, axis=0, keepdims=True)

    @pl.when(i == 0)
    def _():
        out_ref[...] = jnp.zeros_like(out_ref)

    out_ref[...] += part


def _readout1_tc(hn, w1, b1, w2, b2, block):
    n = hn.shape[0]
    return pl.pallas_call(
        _readout1_body,
        grid=(n // block,),
        in_specs=[
            pl.BlockSpec((block, H), lambda i: (i, 0)),
            pl.BlockSpec((H, H), lambda i: (0, 0)),
            pl.BlockSpec((1, H), lambda i: (0, 0)),
            pl.BlockSpec((H, H), lambda i: (0, 0)),
            pl.BlockSpec((1, H), lambda i: (0, 0)),
        ],
        out_specs=pl.BlockSpec((1, H), lambda i: (0, 0)),
        out_shape=jax.ShapeDtypeStruct((1, H), jnp.float32),
    )(hn, w1, b1.reshape(1, H), w2, b2.reshape(1, H))


def _readout2_body(m_ref, w1_ref, b1_ref, w2_ref, b2_ref, out_ref):
    t = _spk(jnp.dot(m_ref[...], w1_ref[...],
                     preferred_element_type=jnp.float32) + b1_ref[...])
    out_ref[...] = jnp.dot(t, w2_ref[...],
                           preferred_element_type=jnp.float32) + b2_ref[...]


def _readout2_tc(m, w1, b1, w2, b2):
    return pl.pallas_call(
        _readout2_body,
        in_specs=[
            pl.BlockSpec((1, H), lambda: (0, 0)),
            pl.BlockSpec((H, H), lambda: (0, 0)),
            pl.BlockSpec((1, H), lambda: (0, 0)),
            pl.BlockSpec((H, 1), lambda: (0, 0)),
            pl.BlockSpec((1, 1), lambda: (0, 0)),
        ],
        out_specs=pl.BlockSpec((1, 1), lambda: (0, 0)),
        out_shape=jax.ShapeDtypeStruct((1, 1), jnp.float32),
    )(m, w1, b1.reshape(1, H), w2, b2.reshape(1, 1))


BLK = 2000


def kernel(r_g, r_h, r_i, emb2, emb3, emb4, conv_W1, conv_b1, conv_W2,
           conv_b2, conv_Wo, conv_bo, fc_W1, fc_b1, fc_W2, fc_b2, fc2_W1,
           fc2_b1, fc2_W2, fc2_b2, edge_index_g, edge_index_h, edge_index_i,
           z):
    n_nodes = z.shape[0]
    eg = r_g.shape[0]
    eh = r_h.shape[0]
    ei = r_i.shape[0]

    gs, gd = edge_index_g[0], edge_index_g[1]
    hs, hd = edge_index_h[0], edge_index_h[1]
    isrc, idst = edge_index_i[0], edge_index_i[1]

    r_g2 = r_g.reshape(eg, 1)
    r_h2 = r_h.reshape(eh, 1)
    r_i2 = r_i.reshape(ei, 1)

    cut_g = _cutoff_tc(r_g2, BLK)
    cut_g1 = cut_g.reshape(eg)
    cut_h1 = jnp.minimum(cut_g1[hs], cut_g1[hd])
    cut_h = cut_h1.reshape(eh, 1)
    cut_i = jnp.minimum(cut_h1[isrc], cut_h1[idst]).reshape(ei, 1)

    eq = (z[gs] == z[gd]).astype(jnp.int32)
    he_g = emb2[eq]
    c1 = z[gs[hs]]
    c2 = z[gd[hs]]
    c3 = z[gd[hd]]
    tbits = ((c1 == c2).astype(jnp.int32) + 2 * (c1 == c3).astype(jnp.int32)
             + 4 * (c2 == c3).astype(jnp.int32))
    he_h = emb3[tbits]
    a = hs[isrc]
    b = hd[isrc]
    d = hd[idst]
    q1 = z[gs[a]]
    q2 = z[gd[a]]
    q3 = z[gd[b]]
    q4 = z[gd[d]]
    qbits = ((q1 == q2).astype(jnp.int32) + 2 * (q1 == q3).astype(jnp.int32)
             + 4 * (q1 == q4).astype(jnp.int32)
             + 8 * (q2 == q3).astype(jnp.int32)
             + 16 * (q2 == q4).astype(jnp.int32)
             + 32 * (q3 == q4).astype(jnp.int32))
    he_i = emb4[qbits]

    hn_g = jnp.ones((n_nodes, H), dtype=r_g.dtype)
    hn_h = he_g
    hn_i = he_h

    for l in range(3):
        e_h = hn_i
        e_g = hn_h
        msg_g = _edge_msg_tc(r_g2, cut_g, hn_g[gs], e_g,
                             conv_W1[l, 0], conv_b1[l, 0], conv_W2[l, 0],
                             conv_b2[l, 0], 0.0, 1.0, BLK)
        agg_g = jax.ops.segment_sum(msg_g, gd, num_segments=n_nodes)
        hn_g_new = _out_update_tc(hn_g, agg_g, conv_Wo[l, 0], conv_bo[l, 0],
                                  BLK)
        msg_h = _edge_msg_tc(r_h2, cut_h, hn_h[hs], e_h,
                             conv_W1[l, 1], conv_b1[l, 1], conv_W2[l, 1],
                             conv_b2[l, 1], -1.0, 1.0, BLK)
        agg_h = jax.ops.segment_sum(msg_h, hd, num_segments=eg)
        hn_h_new = _out_update_tc(hn_h, agg_h, conv_Wo[l, 1], conv_bo[l, 1],
                                  BLK)
        msg_i = _edge_msg_tc(r_i2, cut_i, hn_i[isrc], he_i,
                             conv_W1[l, 2], conv_b1[l, 2], conv_W2[l, 2],
                             conv_b2[l, 2], -1.0, 1.0, BLK)
        agg_i = jax.ops.segment_sum(msg_i, idst, num_segments=eh)
        hn_i_new = _out_update_tc(hn_i, agg_i, conv_Wo[l, 2], conv_bo[l, 2],
                                  BLK)
        hn_g, hn_h, hn_i = hn_g_new, hn_h_new, hn_i_new

    s = _readout1_tc(hn_g, fc_W1, fc_b1, fc_W2, fc_b2, BLK)
    m = s / jnp.float32(n_nodes)
    y = _readout2_tc(m, fc2_W1, fc2_b1, fc2_W2, fc2_b2)
    return y.reshape(1)
